# baseline (speedup 1.0000x reference)
import functools

import jax
import jax.numpy as jnp
from jax.experimental import pallas as pl
from jax.experimental.pallas import tpu as pltpu

N = 100000
D = 128
S = 256
DI = 256
B = 5000
NB = N // B


def _fused_kernel(xu, iu, xv, iv, xy, iy,
                  wgu, bgu, wgv, bgv, wgy, bgy, wnet, bnet,
                  out, accu, accv, accy):
    i = pl.program_id(0)

    @pl.when(i == 0)
    def _init():
        accu[...] = jnp.zeros_like(accu)
        accv[...] = jnp.zeros_like(accv)
        accy[...] = jnp.zeros_like(accy)

    for xref, iref, wg, bg, acc in (
            (xu, iu, wgu, bgu, accu),
            (xv, iv, wgv, bgv, accv),
            (xy, iy, wgy, bgy, accy)):
        xb = xref[...].astype(jnp.bfloat16)
        zh = jnp.dot(xb, wg[...],
                     preferred_element_type=jnp.float32) + bg[...]
        c1 = 0.5 * 1.4426950408889634
        e = jnp.exp2(c1 * jnp.tanh(zh))
        payload = jnp.concatenate([e, e * xref[...]],
                                  axis=1).astype(jnp.bfloat16)
        idx = iref[0]
        onehot_t = (jax.lax.broadcasted_iota(jnp.int32, (S, B), 0)
                    == idx).astype(jnp.bfloat16)
        acc[...] += jnp.dot(onehot_t, payload,
                            preferred_element_type=jnp.float32)

    @pl.when(i == NB - 1)
    def _finish():
        res = bnet[...]
        for k, acc in enumerate((accu, accv, accy)):
            seg_e = acc[:, :D]
            seg_ex = acc[:, D:]
            h = seg_ex / (seg_e + 1e-16)
            res = res + jnp.dot(h, wnet[k],
                                preferred_element_type=jnp.float32)
        out[...] = res


@functools.partial(jax.jit, static_argnames=())
def kernel(x_u, x_v, x_y, index_u, index_v, index_y,
           Wg_u, bg_u, Wg_v, bg_v, Wg_y, bg_y, W_net, b_net):
    iu = index_u.astype(jnp.int32).reshape(NB, 1, B)
    iv = index_v.astype(jnp.int32).reshape(NB, 1, B)
    iy = index_y.astype(jnp.int32).reshape(NB, 1, B)
    wnet = W_net.reshape(3, D, DI)
    wgu = (0.5 * Wg_u).astype(jnp.bfloat16)
    wgv = (0.5 * Wg_v).astype(jnp.bfloat16)
    wgy = (0.5 * Wg_y).astype(jnp.bfloat16)

    x_spec = pl.BlockSpec((B, D), lambda i: (i, 0))
    i_spec = pl.BlockSpec((1, 1, B), lambda i: (i, 0, 0))
    w_spec = pl.BlockSpec((D, D), lambda i: (0, 0))
    b_spec = pl.BlockSpec((1, D), lambda i: (0, 0))

    out = pl.pallas_call(
        _fused_kernel,
        grid=(NB,),
        in_specs=[
            x_spec, i_spec, x_spec, i_spec, x_spec, i_spec,
            w_spec, b_spec, w_spec, b_spec, w_spec, b_spec,
            pl.BlockSpec((3, D, DI), lambda i: (0, 0, 0)),
            pl.BlockSpec((1, DI), lambda i: (0, 0)),
        ],
        out_specs=pl.BlockSpec((S, DI), lambda i: (0, 0)),
        out_shape=jax.ShapeDtypeStruct((S, DI), jnp.float32),
        scratch_shapes=[pltpu.VMEM((S, 2 * D), jnp.float32)] * 3,
    )(x_u, iu, x_v, iv, x_y, iy,
      wgu, (0.5 * bg_u).reshape(1, D), wgv, (0.5 * bg_v).reshape(1, D),
      wgy, (0.5 * bg_y).reshape(1, D), wnet, b_net.reshape(1, DI))
    return out


# speedup vs baseline: 1.0008x; 1.0008x over previous
import functools

import jax
import jax.numpy as jnp
from jax.experimental import pallas as pl
from jax.experimental.pallas import tpu as pltpu

N = 100000
D = 128
S = 256
DI = 256
B = 10000
NB = N // B


def _fused_kernel(xu, iu, xv, iv, xy, iy,
                  wgu, bgu, wgv, bgv, wgy, bgy, wnet, bnet,
                  out, accu, accv, accy):
    i = pl.program_id(0)

    @pl.when(i == 0)
    def _init():
        accu[...] = jnp.zeros_like(accu)
        accv[...] = jnp.zeros_like(accv)
        accy[...] = jnp.zeros_like(accy)

    for xref, iref, wg, bg, acc in (
            (xu, iu, wgu, bgu, accu),
            (xv, iv, wgv, bgv, accv),
            (xy, iy, wgy, bgy, accy)):
        xb = xref[...].astype(jnp.bfloat16)
        zh = jnp.dot(xb, wg[...],
                     preferred_element_type=jnp.float32) + bg[...]
        c1 = 0.5 * 1.4426950408889634
        e = jnp.exp2(c1 * jnp.tanh(zh))
        payload = jnp.concatenate([e, e * xref[...]],
                                  axis=1).astype(jnp.bfloat16)
        idx = iref[0]
        onehot_t = (jax.lax.broadcasted_iota(jnp.int32, (S, B), 0)
                    == idx).astype(jnp.bfloat16)
        acc[...] += jnp.dot(onehot_t, payload,
                            preferred_element_type=jnp.float32)

    @pl.when(i == NB - 1)
    def _finish():
        res = bnet[...]
        for k, acc in enumerate((accu, accv, accy)):
            seg_e = acc[:, :D]
            seg_ex = acc[:, D:]
            h = seg_ex / (seg_e + 1e-16)
            res = res + jnp.dot(h, wnet[k],
                                preferred_element_type=jnp.float32)
        out[...] = res


@functools.partial(jax.jit, static_argnames=())
def kernel(x_u, x_v, x_y, index_u, index_v, index_y,
           Wg_u, bg_u, Wg_v, bg_v, Wg_y, bg_y, W_net, b_net):
    iu = index_u.astype(jnp.int32).reshape(NB, 1, B)
    iv = index_v.astype(jnp.int32).reshape(NB, 1, B)
    iy = index_y.astype(jnp.int32).reshape(NB, 1, B)
    wnet = W_net.reshape(3, D, DI)
    wgu = (0.5 * Wg_u).astype(jnp.bfloat16)
    wgv = (0.5 * Wg_v).astype(jnp.bfloat16)
    wgy = (0.5 * Wg_y).astype(jnp.bfloat16)

    x_spec = pl.BlockSpec((B, D), lambda i: (i, 0))
    i_spec = pl.BlockSpec((1, 1, B), lambda i: (i, 0, 0))
    w_spec = pl.BlockSpec((D, D), lambda i: (0, 0))
    b_spec = pl.BlockSpec((1, D), lambda i: (0, 0))

    out = pl.pallas_call(
        _fused_kernel,
        grid=(NB,),
        in_specs=[
            x_spec, i_spec, x_spec, i_spec, x_spec, i_spec,
            w_spec, b_spec, w_spec, b_spec, w_spec, b_spec,
            pl.BlockSpec((3, D, DI), lambda i: (0, 0, 0)),
            pl.BlockSpec((1, DI), lambda i: (0, 0)),
        ],
        out_specs=pl.BlockSpec((S, DI), lambda i: (0, 0)),
        out_shape=jax.ShapeDtypeStruct((S, DI), jnp.float32),
        scratch_shapes=[pltpu.VMEM((S, 2 * D), jnp.float32)] * 3,
    )(x_u, iu, x_v, iv, x_y, iy,
      wgu, (0.5 * bg_u).reshape(1, D), wgv, (0.5 * bg_v).reshape(1, D),
      wgy, (0.5 * bg_y).reshape(1, D), wnet, b_net.reshape(1, DI))
    return out


# speedup vs baseline: 1.0648x; 1.0640x over previous
import functools

import jax
import jax.numpy as jnp
from jax.experimental import pallas as pl
from jax.experimental.pallas import tpu as pltpu

N = 100000
D = 128
S = 256
DI = 256
B = 10000
NB = N // B


def _fused_kernel(xu, iu, xv, iv, xy, iy,
                  wgu, bgu, wgv, bgv, wgy, bgy, wnet, bnet,
                  out, accu, accv, accy):
    i = pl.program_id(0)

    @pl.when(i == 0)
    def _init():
        accu[...] = jnp.zeros_like(accu)
        accv[...] = jnp.zeros_like(accv)
        accy[...] = jnp.zeros_like(accy)

    for xref, iref, wg, bg, acc in (
            (xu, iu, wgu, bgu, accu),
            (xv, iv, wgv, bgv, accv),
            (xy, iy, wgy, bgy, accy)):
        xb = xref[...].astype(jnp.bfloat16)
        wgb = (wg[...] * 0.5).astype(jnp.bfloat16)
        zh = jnp.dot(xb, wgb,
                     preferred_element_type=jnp.float32) + bg[...]
        c1 = 0.5 * 1.4426950408889634
        e = jnp.exp2(c1 * jnp.tanh(zh))
        payload = jnp.concatenate([e, e * xref[...]],
                                  axis=1).astype(jnp.bfloat16)
        idx = iref[0]
        onehot_t = (jax.lax.broadcasted_iota(jnp.int32, (S, B), 0)
                    == idx).astype(jnp.bfloat16)
        acc[...] += jnp.dot(onehot_t, payload,
                            preferred_element_type=jnp.float32)

    @pl.when(i == NB - 1)
    def _finish():
        res = bnet[...]
        for k, acc in enumerate((accu, accv, accy)):
            seg_e = acc[:, :D]
            seg_ex = acc[:, D:]
            h = seg_ex / (seg_e + 1e-16)
            res = res + jnp.dot(h, wnet[k],
                                preferred_element_type=jnp.float32)
        out[...] = res


@functools.partial(jax.jit, static_argnames=())
def kernel(x_u, x_v, x_y, index_u, index_v, index_y,
           Wg_u, bg_u, Wg_v, bg_v, Wg_y, bg_y, W_net, b_net):
    ii = jnp.concatenate([index_u.astype(jnp.int32),
                          index_v.astype(jnp.int32),
                          index_y.astype(jnp.int32)]).reshape(3 * NB, 1, B)
    wnet = W_net.reshape(3, D, DI)

    x_spec = pl.BlockSpec((B, D), lambda i: (i, 0))
    i_specs = [pl.BlockSpec((1, 1, B), lambda i, p=p: (p * NB + i, 0, 0))
               for p in range(3)]
    w_spec = pl.BlockSpec((D, D), lambda i: (0, 0))
    b_spec = pl.BlockSpec((1, D), lambda i: (0, 0))

    out = pl.pallas_call(
        _fused_kernel,
        grid=(NB,),
        in_specs=[
            x_spec, i_specs[0], x_spec, i_specs[1], x_spec, i_specs[2],
            w_spec, b_spec, w_spec, b_spec, w_spec, b_spec,
            pl.BlockSpec((3, D, DI), lambda i: (0, 0, 0)),
            pl.BlockSpec((1, DI), lambda i: (0, 0)),
        ],
        out_specs=pl.BlockSpec((S, DI), lambda i: (0, 0)),
        out_shape=jax.ShapeDtypeStruct((S, DI), jnp.float32),
        scratch_shapes=[pltpu.VMEM((S, 2 * D), jnp.float32)] * 3,
    )(x_u, ii, x_v, ii, x_y, ii,
      Wg_u, (0.5 * bg_u).reshape(1, D), Wg_v, (0.5 * bg_v).reshape(1, D),
      Wg_y, (0.5 * bg_y).reshape(1, D), wnet, b_net.reshape(1, DI))
    return out
